# baseline (speedup 1.0000x reference)
import functools
import math

import jax
import jax.numpy as jnp
from jax import lax
from jax.experimental import pallas as pl
from jax.experimental.pallas import tpu as pltpu
from jax.experimental.pallas import tpu_sc as plsc

_N_NODES = 10000
_N_EDGES = 160000
_DIM = 128
_N_BASIS = 8
_R_CUT = 5.0

_SC_CORES = 2
_SC_SUBCORES = 16
_GCHUNK = 128

_BLK = 1280
_SUB = 128
_SLICES = (21760, 25600, 30720, 37120, 44800)
_NBLK = 2000


def _pack_bf16_pair(a, b):
    bc = jax.lax.bitcast_convert_type
    ua = bc(a, jnp.uint32)
    ub = bc(b, jnp.uint32)
    lo = (ua + jnp.uint32(0x8000)) >> 16
    hi = (ub + jnp.uint32(0x8000)) & jnp.uint32(0xFFFF0000)
    return bc(lo | hi, jnp.int32)


def _nt_body(n0_ref, nx_ref, ny_ref, nz_ref, U0_ref, U1_ref,
             p0_ref, p1_ref):
    f32 = jnp.float32
    U1 = U1_ref[...]
    h0 = jnp.dot(n0_ref[...], U0_ref[...], preferred_element_type=f32)
    hx = jnp.dot(nx_ref[...], U1, preferred_element_type=f32)
    hy = jnp.dot(ny_ref[...], U1, preferred_element_type=f32)
    hz = jnp.dot(nz_ref[...], U1, preferred_element_type=f32)
    p0_ref[...] = _pack_bf16_pair(h0, hx)
    p1_ref[...] = _pack_bf16_pair(hy, hz)


def _node_transform(n0, nx, ny, nz, U0, U1):
    N, D = n0.shape
    espec = pl.BlockSpec((_NBLK, D), lambda i: (i, 0))
    wspec = pl.BlockSpec((D, D), lambda i: (0, 0))
    oshape = jax.ShapeDtypeStruct((N, D), jnp.int32)
    return pl.pallas_call(
        _nt_body,
        grid=(N // _NBLK,),
        in_specs=[espec, espec, espec, espec, wspec, wspec],
        out_specs=[espec] * 2,
        out_shape=[oshape] * 2,
    )(n0, nx, ny, nz, U0, U1)


def _sc_gather(t0, t1, idx):
    E = idx.shape[0]
    NW = _SC_CORES * _SC_SUBCORES
    n_chunks = E // _GCHUNK
    n_pairs = ((n_chunks + NW - 1) // NW + 1) // 2
    D = _DIM
    G = _GCHUNK
    dt = t0.dtype
    mesh = plsc.VectorSubcoreMesh(core_axis_name="c", subcore_axis_name="s")

    @functools.partial(
        pl.kernel,
        out_type=[jax.ShapeDtypeStruct((E, D), dt)] * 2,
        mesh=mesh,
        scratch_types=[
            pltpu.VMEM((2, G), jnp.int32),
            pltpu.VMEM((2, 2, G, D), dt),
            pltpu.SemaphoreType.DMA,
            pltpu.SemaphoreType.DMA,
        ],
    )
    def gk(t0_h, t1_h, idx_h, o0_h, o1_h, idx_v, rows_v, gsem, ssem):
        wid = lax.axis_index("s") * _SC_CORES + lax.axis_index("c")
        tabs = (t0_h, t1_h)
        outs = (o0_h, o1_h)

        def pair(ip, carry):
            for u in (0, 1):
                ci = wid + (2 * ip + u) * NW

                @pl.when(ci < n_chunks)
                def _(u=u, ci=ci):
                    @pl.when(ip > 0)
                    def _():
                        for p in range(2):
                            pltpu.make_async_copy(
                                rows_v.at[u].at[p],
                                outs[p].at[pl.ds(0, G)], ssem).wait()
                    pltpu.sync_copy(idx_h.at[pl.ds(ci * G, G)], idx_v.at[u])
                    for p in range(2):
                        pltpu.async_copy(tabs[p].at[idx_v.at[u]],
                                         rows_v.at[u].at[p], gsem)
            for u in (0, 1):
                ci = wid + (2 * ip + u) * NW

                @pl.when(ci < n_chunks)
                def _(u=u, ci=ci):
                    for p in range(2):
                        pltpu.make_async_copy(tabs[p].at[idx_v.at[u]],
                                              rows_v.at[u].at[p], gsem).wait()
                    for p in range(2):
                        pltpu.async_copy(rows_v.at[u].at[p],
                                         outs[p].at[pl.ds(ci * G, G)], ssem)
            return carry

        lax.fori_loop(0, n_pairs, pair, 0)
        for u in (0, 1):
            @pl.when(wid + u * NW < n_chunks)
            def _(u=u):
                for p in range(2):
                    pltpu.make_async_copy(rows_v.at[u].at[p],
                                          outs[p].at[pl.ds(0, G)], ssem).wait()

    return gk(t0, t1, idx)


def _tc_body(p0_ref, p1_ref, e0_ref, e1_ref, rijT_ref,
             dij_ref, W_rad_ref, W_nl0_ref, b_nl0_ref,
             W_nl1_ref, b_nl1_ref, I_ref, out0_ref, out1_ref):
    f32 = jnp.float32
    bc = jax.lax.bitcast_convert_type
    B = _SUB
    W_rad = W_rad_ref[...]
    W_nl0 = W_nl0_ref[...]
    W_nl1 = W_nl1_ref[...]
    b_nl0 = b_nl0_ref[...]
    b_nl1 = b_nl1_ref[...]
    I128 = I_ref[...]

    for u in range(_BLK // _SUB):
        lo, hi = u * B, (u + 1) * B

        dij = dij_ref[u]
        t = (math.pi / _R_CUT) ** 2 * (dij * dij)
        cosv = 0.999999443679399 + t * (
            -0.4999955816555435 + t * (
                0.04166103279007576 + t * (
                    -0.0013862747315868196 + t * (
                        2.4253192495892717e-05 + t * -2.2193949937629105e-07))))
        fc = 0.5 * (cosv + 1.0)
        fc = jnp.where(dij < _R_CUT, fc, 0.0)
        dijb = jnp.broadcast_to(dij, (_N_BASIS, B))
        mu = (_R_CUT / (_N_BASIS - 1)) * lax.broadcasted_iota(
            jnp.int32, (_N_BASIS, B), 0).astype(f32)
        rbf = jnp.exp(-4.0 * (dijb - mu) ** 2)
        q = rbf * fc
        fij = jax.lax.dot_general(q, W_rad,
                                  (((0,), (0,)), ((), ())),
                                  preferred_element_type=f32)
        f0 = fij[:, 0 * _DIM:1 * _DIM]
        f1 = fij[:, 1 * _DIM:2 * _DIM]
        f2 = fij[:, 2 * _DIM:3 * _DIM]
        f3 = fij[:, 3 * _DIM:4 * _DIM]

        rxyz = rijT_ref[:, lo:hi]
        rn = jnp.sqrt(jnp.sum(rxyz * rxyz, axis=0, keepdims=True)) + 1e-9
        rhat = rxyz / rn
        rh_em = jax.lax.dot_general(I128, rhat,
                                    (((1,), (1,)), ((), ())),
                                    preferred_element_type=f32)
        rhx = jnp.broadcast_to(rh_em[:, 0:1], (B, _DIM))
        rhy = jnp.broadcast_to(rh_em[:, 1:2], (B, _DIM))
        rhz = jnp.broadcast_to(rh_em[:, 2:3], (B, _DIM))

        w0 = bc(p0_ref[lo:hi, :], jnp.uint32)
        w1 = bc(p1_ref[lo:hi, :], jnp.uint32)
        h0 = bc(w0 << 16, f32)
        h1x = bc(w0 & jnp.uint32(0xFFFF0000), f32)
        h1y = bc(w1 << 16, f32)
        h1z = bc(w1 & jnp.uint32(0xFFFF0000), f32)

        dotr = h1x * rhx + h1y * rhy + h1z * rhz
        m0 = f0 * h0 + f3 * dotr
        g = f1 * h0
        m1x = g * rhx + f2 * h1x
        m1y = g * rhy + f2 * h1y
        m1z = g * rhz + f2 * h1z

        z0 = jnp.dot(m0, W_nl0, preferred_element_type=f32) + b_nl0
        y0 = z0 * (1.0 / (1.0 + jnp.exp(-z0)))
        out0_ref[lo:hi, :] = e0_ref[lo:hi, :] + y0

        norm1 = jnp.sqrt(m1x * m1x + m1y * m1y + m1z * m1z + 1e-9)
        z1 = jnp.dot(norm1, W_nl1, preferred_element_type=f32) + b_nl1
        gate = z1 * (1.0 / (1.0 + jnp.exp(-z1)))
        out1_ref[0, lo:hi, :] = e1_ref[0, lo:hi, :] + m1x * gate
        out1_ref[1, lo:hi, :] = e1_ref[1, lo:hi, :] + m1y * gate
        out1_ref[2, lo:hi, :] = e1_ref[2, lo:hi, :] + m1z * gate


def _tc_call(p0, p1, e0, e1t, rijT, dijr, W_rad,
             W_nl0, b_nl0, W_nl1, b_nl1, I128, base=0, prev=None,
             interpret=False):
    Es = p0.shape[0]
    E = e0.shape[0]
    grid = (Es // _BLK,)
    D = _DIM

    def sb(i):
        return (i, 0)

    def ob(i):
        return (i + base, 0)

    def wb(i):
        return (0, 0)

    sspec = pl.BlockSpec((_BLK, D), sb)
    ospec = pl.BlockSpec((_BLK, D), ob)
    o1spec = pl.BlockSpec((3, _BLK, D), lambda i: (0, i + base, 0))
    in_specs = [
        sspec, sspec, ospec,
        o1spec,
        pl.BlockSpec((3, _BLK), lambda i: (0, i + base)),
        pl.BlockSpec((_BLK // _SUB, 1, _SUB), lambda i: (i + base, 0, 0)),
        pl.BlockSpec((_N_BASIS, 4 * D), wb),
        pl.BlockSpec((D, D), wb),
        pl.BlockSpec((1, D), wb),
        pl.BlockSpec((D, D), wb),
        pl.BlockSpec((1, D), wb),
        pl.BlockSpec((D, D), wb),
    ]
    args = [p0, p1, e0, e1t, rijT, dijr, W_rad,
            W_nl0, b_nl0, W_nl1, b_nl1, I128]
    aliases = {}
    body = _tc_body
    if prev is not None:
        in_specs = in_specs + [
            pl.BlockSpec(memory_space=pltpu.MemorySpace.HBM),
            pl.BlockSpec(memory_space=pltpu.MemorySpace.HBM)]
        args = args + [prev[0], prev[1]]
        aliases = {12: 0, 13: 1}

        def body(*refs):
            _tc_body(*refs[:12], refs[-2], refs[-1])

    out0, out1 = pl.pallas_call(
        body,
        grid=grid,
        in_specs=in_specs,
        out_specs=[ospec, o1spec],
        out_shape=[
            jax.ShapeDtypeStruct((E, D), jnp.float32),
            jax.ShapeDtypeStruct((3, E, D), jnp.float32),
        ],
        input_output_aliases=aliases,
        interpret=interpret,
    )(*args)
    return out0, out1


def kernel(node_info_0, node_info_1, edge_info_0, edge_info_1, edge_index,
           rij, dij, U0, U1, W_rad, W_nl0, b_nl0, W_nl1, b_nl1):
    E = edge_index.shape[1]
    j = edge_index[1].astype(jnp.int32)

    node1t = jnp.transpose(node_info_1, (2, 0, 1))
    P0, P1 = _node_transform(node_info_0, node1t[0], node1t[1],
                             node1t[2], U0, U1)

    e1t = jnp.transpose(edge_info_1, (2, 0, 1))
    rijT = jnp.transpose(rij, (1, 0))
    dijr = dij.reshape(E // _SUB, 1, _SUB)

    I128 = jnp.eye(_DIM, dtype=jnp.float32)
    b0r = b_nl0.reshape(1, _DIM)
    b1r = b_nl1.reshape(1, _DIM)

    sizes = _SLICES
    prev = None
    base_e = 0
    for k, Es in enumerate(sizes):
        jk = lax.slice(j, (base_e,), (base_e + Es,))
        gk0, gk1 = _sc_gather(P0, P1, jk)
        prev = _tc_call(gk0, gk1, edge_info_0, e1t, rijT, dijr,
                        W_rad, W_nl0, b0r, W_nl1, b1r, I128,
                        base=base_e // _BLK, prev=prev)
        base_e += Es
    out0, out1t = prev
    return out0, jnp.transpose(out1t, (1, 2, 0))


# speedup vs baseline: 1.0052x; 1.0052x over previous
import functools
import math

import jax
import jax.numpy as jnp
from jax import lax
from jax.experimental import pallas as pl
from jax.experimental.pallas import tpu as pltpu
from jax.experimental.pallas import tpu_sc as plsc

_N_NODES = 10000
_N_EDGES = 160000
_DIM = 128
_N_BASIS = 8
_R_CUT = 5.0

_SC_CORES = 2
_SC_SUBCORES = 16
_GCHUNK = 128

_BLK = 1280
_SUB = 128
_SLICES = (12800, 25600, 38400, 40960, 42240)
_NBLK = 2000


def _pack_bf16_pair(a, b):
    bc = jax.lax.bitcast_convert_type
    ua = bc(a, jnp.uint32)
    ub = bc(b, jnp.uint32)
    lo = (ua + jnp.uint32(0x8000)) >> 16
    hi = (ub + jnp.uint32(0x8000)) & jnp.uint32(0xFFFF0000)
    return bc(lo | hi, jnp.int32)


def _nt_body(n0_ref, nx_ref, ny_ref, nz_ref, U0_ref, U1_ref,
             p0_ref, p1_ref):
    f32 = jnp.float32
    U1 = U1_ref[...]
    h0 = jnp.dot(n0_ref[...], U0_ref[...], preferred_element_type=f32)
    hx = jnp.dot(nx_ref[...], U1, preferred_element_type=f32)
    hy = jnp.dot(ny_ref[...], U1, preferred_element_type=f32)
    hz = jnp.dot(nz_ref[...], U1, preferred_element_type=f32)
    p0_ref[...] = _pack_bf16_pair(h0, hx)
    p1_ref[...] = _pack_bf16_pair(hy, hz)


def _node_transform(n0, nx, ny, nz, U0, U1):
    N, D = n0.shape
    espec = pl.BlockSpec((_NBLK, D), lambda i: (i, 0))
    wspec = pl.BlockSpec((D, D), lambda i: (0, 0))
    oshape = jax.ShapeDtypeStruct((N, D), jnp.int32)
    return pl.pallas_call(
        _nt_body,
        grid=(N // _NBLK,),
        in_specs=[espec, espec, espec, espec, wspec, wspec],
        out_specs=[espec] * 2,
        out_shape=[oshape] * 2,
    )(n0, nx, ny, nz, U0, U1)


def _sc_gather(t0, t1, idx):
    E = idx.shape[0]
    NW = _SC_CORES * _SC_SUBCORES
    n_chunks = E // _GCHUNK
    n_pairs = ((n_chunks + NW - 1) // NW + 1) // 2
    D = _DIM
    G = _GCHUNK
    dt = t0.dtype
    mesh = plsc.VectorSubcoreMesh(core_axis_name="c", subcore_axis_name="s")

    @functools.partial(
        pl.kernel,
        out_type=[jax.ShapeDtypeStruct((E, D), dt)] * 2,
        mesh=mesh,
        scratch_types=[
            pltpu.VMEM((2, G), jnp.int32),
            pltpu.VMEM((2, 2, G, D), dt),
            pltpu.SemaphoreType.DMA,
            pltpu.SemaphoreType.DMA,
        ],
    )
    def gk(t0_h, t1_h, idx_h, o0_h, o1_h, idx_v, rows_v, gsem, ssem):
        wid = lax.axis_index("s") * _SC_CORES + lax.axis_index("c")
        tabs = (t0_h, t1_h)
        outs = (o0_h, o1_h)

        def pair(ip, carry):
            for u in (0, 1):
                ci = wid + (2 * ip + u) * NW

                @pl.when(ci < n_chunks)
                def _(u=u, ci=ci):
                    @pl.when(ip > 0)
                    def _():
                        for p in range(2):
                            pltpu.make_async_copy(
                                rows_v.at[u].at[p],
                                outs[p].at[pl.ds(0, G)], ssem).wait()
                    pltpu.sync_copy(idx_h.at[pl.ds(ci * G, G)], idx_v.at[u])
                    for p in range(2):
                        pltpu.async_copy(tabs[p].at[idx_v.at[u]],
                                         rows_v.at[u].at[p], gsem)
            for u in (0, 1):
                ci = wid + (2 * ip + u) * NW

                @pl.when(ci < n_chunks)
                def _(u=u, ci=ci):
                    for p in range(2):
                        pltpu.make_async_copy(tabs[p].at[idx_v.at[u]],
                                              rows_v.at[u].at[p], gsem).wait()
                    for p in range(2):
                        pltpu.async_copy(rows_v.at[u].at[p],
                                         outs[p].at[pl.ds(ci * G, G)], ssem)
            return carry

        lax.fori_loop(0, n_pairs, pair, 0)
        for u in (0, 1):
            @pl.when(wid + u * NW < n_chunks)
            def _(u=u):
                for p in range(2):
                    pltpu.make_async_copy(rows_v.at[u].at[p],
                                          outs[p].at[pl.ds(0, G)], ssem).wait()

    return gk(t0, t1, idx)


def _tc_body(p0_ref, p1_ref, e0_ref, e1_ref, rijT_ref,
             dij_ref, W_rad_ref, W_nl0_ref, b_nl0_ref,
             W_nl1_ref, b_nl1_ref, I_ref, out0_ref, out1_ref):
    f32 = jnp.float32
    bc = jax.lax.bitcast_convert_type
    B = _SUB
    W_rad = W_rad_ref[...]
    W_nl0 = W_nl0_ref[...]
    W_nl1 = W_nl1_ref[...]
    b_nl0 = b_nl0_ref[...]
    b_nl1 = b_nl1_ref[...]
    I128 = I_ref[...]

    for u in range(_BLK // _SUB):
        lo, hi = u * B, (u + 1) * B

        dij = dij_ref[u]
        t = (math.pi / _R_CUT) ** 2 * (dij * dij)
        cosv = 0.999999443679399 + t * (
            -0.4999955816555435 + t * (
                0.04166103279007576 + t * (
                    -0.0013862747315868196 + t * (
                        2.4253192495892717e-05 + t * -2.2193949937629105e-07))))
        fc = 0.5 * (cosv + 1.0)
        fc = jnp.where(dij < _R_CUT, fc, 0.0)
        dijb = jnp.broadcast_to(dij, (_N_BASIS, B))
        mu = (_R_CUT / (_N_BASIS - 1)) * lax.broadcasted_iota(
            jnp.int32, (_N_BASIS, B), 0).astype(f32)
        rbf = jnp.exp(-4.0 * (dijb - mu) ** 2)
        q = rbf * fc
        fij = jax.lax.dot_general(q, W_rad,
                                  (((0,), (0,)), ((), ())),
                                  preferred_element_type=f32)
        f0 = fij[:, 0 * _DIM:1 * _DIM]
        f1 = fij[:, 1 * _DIM:2 * _DIM]
        f2 = fij[:, 2 * _DIM:3 * _DIM]
        f3 = fij[:, 3 * _DIM:4 * _DIM]

        rxyz = rijT_ref[:, lo:hi]
        rn = jnp.sqrt(jnp.sum(rxyz * rxyz, axis=0, keepdims=True)) + 1e-9
        rhat = rxyz / rn
        rh_em = jax.lax.dot_general(I128, rhat,
                                    (((1,), (1,)), ((), ())),
                                    preferred_element_type=f32)
        rhx = jnp.broadcast_to(rh_em[:, 0:1], (B, _DIM))
        rhy = jnp.broadcast_to(rh_em[:, 1:2], (B, _DIM))
        rhz = jnp.broadcast_to(rh_em[:, 2:3], (B, _DIM))

        w0 = bc(p0_ref[lo:hi, :], jnp.uint32)
        w1 = bc(p1_ref[lo:hi, :], jnp.uint32)
        h0 = bc(w0 << 16, f32)
        h1x = bc(w0 & jnp.uint32(0xFFFF0000), f32)
        h1y = bc(w1 << 16, f32)
        h1z = bc(w1 & jnp.uint32(0xFFFF0000), f32)

        dotr = h1x * rhx + h1y * rhy + h1z * rhz
        m0 = f0 * h0 + f3 * dotr
        g = f1 * h0
        m1x = g * rhx + f2 * h1x
        m1y = g * rhy + f2 * h1y
        m1z = g * rhz + f2 * h1z

        z0 = jnp.dot(m0, W_nl0, preferred_element_type=f32) + b_nl0
        y0 = z0 * (1.0 / (1.0 + jnp.exp(-z0)))
        out0_ref[lo:hi, :] = e0_ref[lo:hi, :] + y0

        norm1 = jnp.sqrt(m1x * m1x + m1y * m1y + m1z * m1z + 1e-9)
        z1 = jnp.dot(norm1, W_nl1, preferred_element_type=f32) + b_nl1
        gate = z1 * (1.0 / (1.0 + jnp.exp(-z1)))
        out1_ref[0, lo:hi, :] = e1_ref[0, lo:hi, :] + m1x * gate
        out1_ref[1, lo:hi, :] = e1_ref[1, lo:hi, :] + m1y * gate
        out1_ref[2, lo:hi, :] = e1_ref[2, lo:hi, :] + m1z * gate


def _tc_call(p0, p1, e0, e1t, rijT, dijr, W_rad,
             W_nl0, b_nl0, W_nl1, b_nl1, I128, base=0, prev=None,
             interpret=False):
    Es = p0.shape[0]
    E = e0.shape[0]
    grid = (Es // _BLK,)
    D = _DIM

    def sb(i):
        return (i, 0)

    def ob(i):
        return (i + base, 0)

    def wb(i):
        return (0, 0)

    sspec = pl.BlockSpec((_BLK, D), sb)
    ospec = pl.BlockSpec((_BLK, D), ob)
    o1spec = pl.BlockSpec((3, _BLK, D), lambda i: (0, i + base, 0))
    in_specs = [
        sspec, sspec, ospec,
        o1spec,
        pl.BlockSpec((3, _BLK), lambda i: (0, i + base)),
        pl.BlockSpec((_BLK // _SUB, 1, _SUB), lambda i: (i + base, 0, 0)),
        pl.BlockSpec((_N_BASIS, 4 * D), wb),
        pl.BlockSpec((D, D), wb),
        pl.BlockSpec((1, D), wb),
        pl.BlockSpec((D, D), wb),
        pl.BlockSpec((1, D), wb),
        pl.BlockSpec((D, D), wb),
    ]
    args = [p0, p1, e0, e1t, rijT, dijr, W_rad,
            W_nl0, b_nl0, W_nl1, b_nl1, I128]
    aliases = {}
    body = _tc_body
    if prev is not None:
        in_specs = in_specs + [
            pl.BlockSpec(memory_space=pltpu.MemorySpace.HBM),
            pl.BlockSpec(memory_space=pltpu.MemorySpace.HBM)]
        args = args + [prev[0], prev[1]]
        aliases = {12: 0, 13: 1}

        def body(*refs):
            _tc_body(*refs[:12], refs[-2], refs[-1])

    out0, out1 = pl.pallas_call(
        body,
        grid=grid,
        in_specs=in_specs,
        out_specs=[ospec, o1spec],
        out_shape=[
            jax.ShapeDtypeStruct((E, D), jnp.float32),
            jax.ShapeDtypeStruct((3, E, D), jnp.float32),
        ],
        input_output_aliases=aliases,
        interpret=interpret,
    )(*args)
    return out0, out1


def kernel(node_info_0, node_info_1, edge_info_0, edge_info_1, edge_index,
           rij, dij, U0, U1, W_rad, W_nl0, b_nl0, W_nl1, b_nl1):
    E = edge_index.shape[1]
    j = edge_index[1].astype(jnp.int32)

    node1t = jnp.transpose(node_info_1, (2, 0, 1))
    P0, P1 = _node_transform(node_info_0, node1t[0], node1t[1],
                             node1t[2], U0, U1)

    e1t = jnp.transpose(edge_info_1, (2, 0, 1))
    rijT = jnp.transpose(rij, (1, 0))
    dijr = dij.reshape(E // _SUB, 1, _SUB)

    I128 = jnp.eye(_DIM, dtype=jnp.float32)
    b0r = b_nl0.reshape(1, _DIM)
    b1r = b_nl1.reshape(1, _DIM)

    sizes = _SLICES
    prev = None
    base_e = 0
    for k, Es in enumerate(sizes):
        jk = lax.slice(j, (base_e,), (base_e + Es,))
        gk0, gk1 = _sc_gather(P0, P1, jk)
        prev = _tc_call(gk0, gk1, edge_info_0, e1t, rijT, dijr,
                        W_rad, W_nl0, b0r, W_nl1, b1r, I128,
                        base=base_e // _BLK, prev=prev)
        base_e += Es
    out0, out1t = prev
    return out0, jnp.transpose(out1t, (1, 2, 0))
